# baseline (speedup 1.0000x reference)
import functools

import jax
import jax.numpy as jnp
from jax.experimental import pallas as pl

N, D, H, E = 32768, 768, 768, 8
BLK = 1024


def _moe_block(x_ref, wg_ref, bg_ref, we_ref, be_ref, out_ref):
    xb16 = x_ref[...]
    logits = jnp.dot(xb16, wg_ref[...],
                     preferred_element_type=jnp.float32) + bg_ref[...]

    m = jnp.max(logits, axis=-1, keepdims=True)
    ex = jnp.exp(logits - m)
    gw = ex / jnp.sum(ex, axis=-1, keepdims=True)

    iota = jax.lax.broadcasted_iota(jnp.int32, gw.shape, 1)
    w1 = jnp.max(gw, axis=-1, keepdims=True)
    i1 = jnp.min(jnp.where(gw == w1, iota, E), axis=-1, keepdims=True)
    sel1 = iota == i1
    w_rest = jnp.where(sel1, -jnp.inf, gw)
    w2 = jnp.max(w_rest, axis=-1, keepdims=True)
    i2 = jnp.min(jnp.where(w_rest == w2, iota, E), axis=-1, keepdims=True)
    sel2 = iota == i2
    g = jnp.where(sel1 | sel2, gw, 0.0)

    acc = jnp.dot(g, be_ref[...], preferred_element_type=jnp.float32)
    for e in range(E):
        ye = jnp.dot(xb16, we_ref[e], preferred_element_type=jnp.float32)
        acc += g[:, e:e + 1] * ye
    out_ref[...] = acc


@functools.partial(jax.jit, static_argnames=())
def kernel(x, Wg, bg, We, be):
    x16 = x.astype(jnp.bfloat16)
    wg16 = Wg.astype(jnp.bfloat16)
    we16 = We.astype(jnp.bfloat16)
    bg2 = bg.reshape(1, E)
    grid = (N // BLK,)
    return pl.pallas_call(
        _moe_block,
        grid=grid,
        in_specs=[
            pl.BlockSpec((BLK, D), lambda i: (i, 0)),
            pl.BlockSpec((D, E), lambda i: (0, 0)),
            pl.BlockSpec((1, E), lambda i: (0, 0)),
            pl.BlockSpec((E, D, H), lambda i: (0, 0, 0)),
            pl.BlockSpec((E, H), lambda i: (0, 0)),
        ],
        out_specs=pl.BlockSpec((BLK, H), lambda i: (i, 0)),
        out_shape=jax.ShapeDtypeStruct((N, H), jnp.float32),
    )(x16, wg16, bg2, we16, be)


# speedup vs baseline: 1.1179x; 1.1179x over previous
import functools

import jax
import jax.numpy as jnp
from jax.experimental import pallas as pl

N, D, H, E = 32768, 768, 768, 8
BLK = 1024


def _moe_block(x_ref, wg_ref, bg_ref, we_ref, be_ref, out_ref):
    xb16 = x_ref[...].astype(jnp.bfloat16)
    logits = jnp.dot(xb16, wg_ref[...],
                     preferred_element_type=jnp.float32) + bg_ref[...]

    m = jnp.max(logits, axis=-1, keepdims=True)
    ex = jnp.exp(logits - m)
    gw = ex / jnp.sum(ex, axis=-1, keepdims=True)

    iota = jax.lax.broadcasted_iota(jnp.int32, gw.shape, 1)
    w1 = jnp.max(gw, axis=-1, keepdims=True)
    i1 = jnp.min(jnp.where(gw == w1, iota, E), axis=-1, keepdims=True)
    sel1 = iota == i1
    w_rest = jnp.where(sel1, -jnp.inf, gw)
    w2 = jnp.max(w_rest, axis=-1, keepdims=True)
    i2 = jnp.min(jnp.where(w_rest == w2, iota, E), axis=-1, keepdims=True)
    sel2 = iota == i2
    g = jnp.where(sel1 | sel2, gw, 0.0)

    acc = jnp.dot(g, be_ref[...], preferred_element_type=jnp.float32)
    for e in range(E):
        ye = jnp.dot(xb16, we_ref[e], preferred_element_type=jnp.float32)
        acc += g[:, e:e + 1] * ye
    out_ref[...] = acc


@functools.partial(jax.jit, static_argnames=())
def kernel(x, Wg, bg, We, be):
    wg16 = Wg.astype(jnp.bfloat16)
    we16 = We.astype(jnp.bfloat16)
    bg2 = bg.reshape(1, E)
    grid = (N // BLK,)
    return pl.pallas_call(
        _moe_block,
        grid=grid,
        in_specs=[
            pl.BlockSpec((BLK, D), lambda i: (i, 0)),
            pl.BlockSpec((D, E), lambda i: (0, 0)),
            pl.BlockSpec((1, E), lambda i: (0, 0)),
            pl.BlockSpec((E, D, H), lambda i: (0, 0, 0)),
            pl.BlockSpec((E, H), lambda i: (0, 0)),
        ],
        out_specs=pl.BlockSpec((BLK, H), lambda i: (i, 0)),
        out_shape=jax.ShapeDtypeStruct((N, H), jnp.float32),
    )(x, wg16, bg2, we16, be)
